# baseline (speedup 1.0000x reference)
import functools

import jax
import jax.numpy as jnp
from jax import lax
from jax.experimental import pallas as pl
from jax.experimental.pallas import tpu as pltpu
from jax.experimental.pallas import tpu_sc as plsc

N = 8192
D = 4096
L = 16
NC = 2
NS = 16
NW = NC * NS
ROWS_PER_W = N // NW
RB = 8
NB = ROWS_PER_W // RB
NQ = 4
DQ = D // NQ
JC_Q = DQ // L


def _body(x_hbm, perm_hbm, out_hbm, perm_v, in0, in1, ob0, ob1, spmem,
          sin0, sin1, sx0, sx1, sd0, sd1):
    wid = lax.axis_index("s") * NC + lax.axis_index("c")
    sid = lax.axis_index("s")
    row_base = wid * ROWS_PER_W

    ins = [in0, in1]
    obs = [ob0, ob1]
    sins = [sin0, sin1]
    sxs = [sx0, sx1]
    sds = [sd0, sd1]

    pltpu.sync_copy(perm_hbm, perm_v)

    row_splats = [jnp.full((L,), r, dtype=jnp.int32) for r in range(RB)]

    def start_in(b, p):
        pltpu.async_copy(
            x_hbm.at[pl.ds(row_base + b * RB, RB)], ins[p], sins[p])

    def wait_in(b, p):
        pltpu.make_async_copy(
            x_hbm.at[pl.ds(row_base + b * RB, RB)], ins[p], sins[p]).wait()

    def start_x(sl):
        pltpu.async_copy(obs[sl], spmem.at[sid, sl], sxs[sl])

    def wait_x(sl):
        pltpu.make_async_copy(obs[sl], spmem.at[sid, sl], sxs[sl]).wait()

    def start_d(b, q):
        sl = q % 2
        pltpu.async_copy(
            spmem.at[sid, sl],
            out_hbm.at[pl.ds(row_base + b * RB, RB), pl.ds(q * DQ, DQ)],
            sds[sl])

    def wait_d(b, q):
        sl = q % 2
        pltpu.make_async_copy(
            spmem.at[sid, sl],
            out_hbm.at[pl.ds(row_base + b * RB, RB), pl.ds(q * DQ, DQ)],
            sds[sl]).wait()

    def compute_quarter(p, q):
        in_b = ins[p]
        out_b = obs[q % 2]

        @plsc.parallel_loop(0, JC_Q, unroll=4)
        def j_body(j):
            idxs = perm_v[pl.ds((q * JC_Q + j) * L, L)]
            for r in range(RB):
                vals = plsc.load_gather(in_b, [row_splats[r], idxs])
                out_b[r, pl.ds(j * L, L)] = vals

    start_in(0, 0)

    def pair_body(pair, carry):
        for p in range(2):
            b = pair * 2 + p

            @pl.when(b + 1 < NB)
            def _():
                start_in(b + 1, 1 - p)

            wait_in(b, p)
            for q in range(NQ):
                sl = q % 2
                if q >= 1:
                    wait_x((q - 1) % 2)
                    start_d(b, q - 1)
                else:
                    @pl.when(b >= 1)
                    def _():
                        wait_x(1)
                        start_d(b - 1, NQ - 1)
                if q >= 2:
                    wait_d(b, q - 2)
                else:
                    @pl.when(b >= 1)
                    def _():
                        wait_d(b - 1, q + 2)

                compute_quarter(p, q)
                start_x(sl)
        return carry

    lax.fori_loop(0, NB // 2, pair_body, 0)
    wait_x(1)
    start_d(NB - 1, NQ - 1)
    wait_d(NB - 1, NQ - 2)
    wait_d(NB - 1, NQ - 1)


@jax.jit
def _permute_cols(x, perm32):
    mesh = plsc.VectorSubcoreMesh(core_axis_name="c", subcore_axis_name="s")
    kern = functools.partial(
        pl.kernel,
        mesh=mesh,
        out_type=jax.ShapeDtypeStruct((N, D), jnp.float32),
        compiler_params=pltpu.CompilerParams(needs_layout_passes=False),
        scratch_types=[
            pltpu.VMEM((D,), jnp.int32),
            pltpu.VMEM((RB, D), jnp.float32),
            pltpu.VMEM((RB, D), jnp.float32),
            pltpu.VMEM((RB, DQ), jnp.float32),
            pltpu.VMEM((RB, DQ), jnp.float32),
            pltpu.VMEM_SHARED((NS, 2, RB, DQ), jnp.float32),
            pltpu.SemaphoreType.DMA,
            pltpu.SemaphoreType.DMA,
            pltpu.SemaphoreType.DMA,
            pltpu.SemaphoreType.DMA,
            pltpu.SemaphoreType.DMA,
            pltpu.SemaphoreType.DMA,
        ],
    )(_body)
    return kern(x, perm32)


def kernel(x, permutation):
    perm32 = permutation.astype(jnp.int32)
    return _permute_cols(x, perm32)
